# baseline (speedup 1.0000x reference)
import functools

import jax
import jax.numpy as jnp
from jax import lax
from jax.experimental import pallas as pl
from jax.experimental.pallas import tpu as pltpu
from jax.experimental.pallas import tpu_sc as plsc

NUM_FRAMES = 100000
B = 16384
NC = 2
NS = 16
NW = NC * NS
BPW = B // NW
CH = 128
NCH = BPW // CH
L = 16
PERIOD = 80

DN = 3
DB = 69
FAT = 128
NV = BPW * DN
BV = BPW * DB

FBLK = 2048


def _dg(v, idx):
    dn = lax.GatherDimensionNumbers(
        offset_dims=(), collapsed_slice_dims=(0,), start_index_map=(0,))
    return lax.gather(v, idx[:, None], dn, slice_sizes=(1,),
                      mode=lax.GatherScatterMode.PROMISE_IN_BOUNDS)


def _pack_body(bp_ref, go_ref, tr_ref, out_ref):
    out_ref[:, 0:DB] = bp_ref[...].T
    out_ref[:, DB:DB + DN] = go_ref[...].T
    out_ref[:, DB + DN:DB + 2 * DN] = tr_ref[...].T


def _pack(bp_t, go_t, tr_t):
    grid = (NUM_FRAMES + FBLK - 1) // FBLK
    return pl.pallas_call(
        _pack_body,
        grid=(grid,),
        in_specs=[
            pl.BlockSpec((DB, FBLK), lambda n: (0, n)),
            pl.BlockSpec((DN, FBLK), lambda n: (0, n)),
            pl.BlockSpec((DN, FBLK), lambda n: (0, n)),
        ],
        out_specs=pl.BlockSpec((FBLK, FAT), lambda n: (n, 0)),
        out_shape=jax.ShapeDtypeStruct((NUM_FRAMES, FAT), jnp.float32),
    )(bp_t, go_t, tr_t)


def _body(idx_hbm, betas_hbm, fat_hbm,
          betas_out, go_out, tr_out, bp_out,
          idx_v, win0, win1, go_v, tr_v, bp_v, beta_row, beta_v,
          sem0, sem1):
    c = lax.axis_index("c")
    s = lax.axis_index("s")
    wid = s * NC + c
    lane = lax.iota(jnp.int32, L)

    pltpu.sync_copy(idx_hbm.at[pl.ds(wid * NCH, NCH)], idx_v)

    wins = (win0, win1)
    sems = (sem0, sem1)

    def fire(j):
        return pltpu.async_copy(fat_hbm.at[idx_v.at[j]], wins[j & 1],
                                sems[j & 1])

    inflight = fire(0)

    pltpu.sync_copy(betas_hbm, beta_row.at[pl.ds(0, 10)])
    row_v = beta_row[...]
    vregs = [_dg(row_v, lax.rem(lane + (k * L) % 10, 10))
             for k in range(PERIOD // L)]

    def bfill(i, carry):
        off = i * PERIOD
        for k in range(PERIOD // L):
            beta_v[pl.ds(off + k * L, L)] = vregs[k]
        return carry

    lax.fori_loop(0, BPW * 10 // PERIOD, bfill, 0)
    pltpu.sync_copy(beta_v, betas_out.at[pl.ds(wid * BPW * 10, BPW * 10)])

    rix_go = (lane + DB - 4 * L) & (L - 1)
    rix_tr = (lane + DB + DN - 4 * L) & (L - 1)

    for j in range(NCH):
        nxt = fire(j + 1) if j + 1 < NCH else None
        inflight.wait()
        inflight = nxt
        win = wins[j & 1]

        def ext(t, carry):
            k3 = DN * (j * CH) + DN * t
            k69 = DB * (j * CH) + DB * t
            w4 = win[t, pl.ds(4 * L, L)]
            for m in range(4):
                bp_v[pl.ds(k69 + m * L, L)] = win[t, pl.ds(m * L, L)]
            bp_v[pl.ds(k69 + 4 * L, L)] = w4
            go_v[pl.ds(k3, L)] = _dg(w4, rix_go)
            tr_v[pl.ds(k3, L)] = _dg(w4, rix_tr)
            return carry

        lax.fori_loop(0, CH, ext, 0)

    pltpu.sync_copy(go_v.at[pl.ds(0, NV)], go_out.at[pl.ds(wid * NV, NV)])
    pltpu.sync_copy(tr_v.at[pl.ds(0, NV)], tr_out.at[pl.ds(wid * NV, NV)])
    pltpu.sync_copy(bp_v.at[pl.ds(0, BV)], bp_out.at[pl.ds(wid * BV, BV)])


@jax.jit
def _run(idx2, betas_w, fat):
    mesh = plsc.VectorSubcoreMesh(core_axis_name="c", subcore_axis_name="s")
    f = pl.kernel(
        _body,
        mesh=mesh,
        compiler_params=pltpu.CompilerParams(use_tc_tiling_on_sc=False),
        out_type=(
            jax.ShapeDtypeStruct((B * 10,), jnp.float32),
            jax.ShapeDtypeStruct((B * DN,), jnp.float32),
            jax.ShapeDtypeStruct((B * DN,), jnp.float32),
            jax.ShapeDtypeStruct((B * DB,), jnp.float32),
        ),
        scratch_types=[
            pltpu.VMEM((NCH, CH), jnp.int32),
            pltpu.VMEM((CH, FAT), jnp.float32),
            pltpu.VMEM((CH, FAT), jnp.float32),
            pltpu.VMEM((NV + L,), jnp.float32),
            pltpu.VMEM((NV + L,), jnp.float32),
            pltpu.VMEM((BV + L,), jnp.float32),
            pltpu.VMEM((L,), jnp.float32),
            pltpu.VMEM((BPW * 10,), jnp.float32),
            pltpu.SemaphoreType.DMA,
            pltpu.SemaphoreType.DMA,
        ],
    )
    return f(idx2, betas_w, fat)


def kernel(frame_ids, betas_w, global_orient_w, transl_w, body_pose_w):
    idx2 = frame_ids.astype(jnp.int32).reshape(NW * NCH, CH)
    fat = jnp.concatenate(
        [body_pose_w, global_orient_w, transl_w,
         jnp.zeros((NUM_FRAMES, FAT - DB - 2 * DN), jnp.float32)], axis=1)
    betas_f, go_f, tr_f, bp_f = _run(idx2, betas_w.reshape(10), fat)
    return (betas_f.reshape(B, 10), go_f.reshape(B, DN),
            tr_f.reshape(B, DN), bp_f.reshape(B, DB))


# speedup vs baseline: 1.9368x; 1.9368x over previous
import functools

import jax
import jax.numpy as jnp
from jax import lax
from jax.experimental import pallas as pl
from jax.experimental.pallas import tpu as pltpu
from jax.experimental.pallas import tpu_sc as plsc

NUM_FRAMES = 100000
B = 16384
NC = 2
NS = 16
NW = NC * NS
BPW = B // NW
CH = 128
NCH = BPW // CH
L = 16
PERIOD = 80

DN = 3
DB = 69
FAT = 128
NV = BPW * DN
BV = BPW * DB

FBLK = 4096


def _dg(v, idx):
    dn = lax.GatherDimensionNumbers(
        offset_dims=(), collapsed_slice_dims=(0,), start_index_map=(0,))
    return lax.gather(v, idx[:, None], dn, slice_sizes=(1,),
                      mode=lax.GatherScatterMode.PROMISE_IN_BOUNDS)


def _pack_body(bp_ref, go_ref, tr_ref, out_ref):
    out_ref[:, 0:DB] = bp_ref[...].T
    gt = jnp.concatenate([go_ref[...], tr_ref[...]], axis=0)
    out_ref[:, DB:DB + 2 * DN] = gt.T


def _pack(bp_t, go_t, tr_t):
    grid = (NUM_FRAMES + FBLK - 1) // FBLK
    return pl.pallas_call(
        _pack_body,
        grid=(grid,),
        in_specs=[
            pl.BlockSpec((DB, FBLK), lambda n: (0, n)),
            pl.BlockSpec((DN, FBLK), lambda n: (0, n)),
            pl.BlockSpec((DN, FBLK), lambda n: (0, n)),
        ],
        out_specs=pl.BlockSpec((FBLK, FAT), lambda n: (n, 0)),
        out_shape=jax.ShapeDtypeStruct((NUM_FRAMES, FAT), jnp.float32),
    )(bp_t, go_t, tr_t)


def _body(idx_hbm, betas_hbm, fat_hbm,
          betas_out, go_out, tr_out, bp_out,
          idx_v, win0, win1, go_v, tr_v, bp_v, beta_row, beta_v,
          sem0, sem1):
    c = lax.axis_index("c")
    s = lax.axis_index("s")
    wid = s * NC + c
    lane = lax.iota(jnp.int32, L)

    pltpu.sync_copy(idx_hbm.at[pl.ds(wid * NCH, NCH)], idx_v)

    wins = (win0, win1)
    sems = (sem0, sem1)

    def fire(j):
        return pltpu.async_copy(fat_hbm.at[idx_v.at[j]], wins[j & 1],
                                sems[j & 1])

    inflight = fire(0)

    pltpu.sync_copy(betas_hbm, beta_row.at[pl.ds(0, 10)])
    row_v = beta_row[...]
    vregs = [_dg(row_v, lax.rem(lane + (k * L) % 10, 10))
             for k in range(PERIOD // L)]

    def bfill(i, carry):
        off = i * PERIOD
        for k in range(PERIOD // L):
            beta_v[pl.ds(off + k * L, L)] = vregs[k]
        return carry

    lax.fori_loop(0, BPW * 10 // PERIOD, bfill, 0)
    pltpu.sync_copy(beta_v, betas_out.at[pl.ds(wid * BPW * 10, BPW * 10)])

    rix_go = (lane + DB - 4 * L) & (L - 1)
    rix_tr = (lane + DB + DN - 4 * L) & (L - 1)

    for j in range(NCH):
        nxt = fire(j + 1) if j + 1 < NCH else None
        inflight.wait()
        inflight = nxt
        win = wins[j & 1]

        def ext(t, carry):
            k3 = DN * (j * CH) + DN * t
            k69 = DB * (j * CH) + DB * t
            w4 = win[t, pl.ds(4 * L, L)]
            for m in range(4):
                bp_v[pl.ds(k69 + m * L, L)] = win[t, pl.ds(m * L, L)]
            bp_v[pl.ds(k69 + 4 * L, L)] = w4
            go_v[pl.ds(k3, L)] = _dg(w4, rix_go)
            tr_v[pl.ds(k3, L)] = _dg(w4, rix_tr)
            return carry

        lax.fori_loop(0, CH, ext, 0)

    pltpu.sync_copy(go_v.at[pl.ds(0, NV)], go_out.at[pl.ds(wid * NV, NV)])
    pltpu.sync_copy(tr_v.at[pl.ds(0, NV)], tr_out.at[pl.ds(wid * NV, NV)])
    pltpu.sync_copy(bp_v.at[pl.ds(0, BV)], bp_out.at[pl.ds(wid * BV, BV)])


@jax.jit
def _run(idx2, betas_w, fat):
    mesh = plsc.VectorSubcoreMesh(core_axis_name="c", subcore_axis_name="s")
    f = pl.kernel(
        _body,
        mesh=mesh,
        compiler_params=pltpu.CompilerParams(use_tc_tiling_on_sc=False),
        out_type=(
            jax.ShapeDtypeStruct((B * 10,), jnp.float32),
            jax.ShapeDtypeStruct((B * DN,), jnp.float32),
            jax.ShapeDtypeStruct((B * DN,), jnp.float32),
            jax.ShapeDtypeStruct((B * DB,), jnp.float32),
        ),
        scratch_types=[
            pltpu.VMEM((NCH, CH), jnp.int32),
            pltpu.VMEM((CH, FAT), jnp.float32),
            pltpu.VMEM((CH, FAT), jnp.float32),
            pltpu.VMEM((NV + L,), jnp.float32),
            pltpu.VMEM((NV + L,), jnp.float32),
            pltpu.VMEM((BV + L,), jnp.float32),
            pltpu.VMEM((L,), jnp.float32),
            pltpu.VMEM((BPW * 10,), jnp.float32),
            pltpu.SemaphoreType.DMA,
            pltpu.SemaphoreType.DMA,
        ],
    )
    return f(idx2, betas_w, fat)


def kernel(frame_ids, betas_w, global_orient_w, transl_w, body_pose_w):
    idx2 = frame_ids.astype(jnp.int32).reshape(NW * NCH, CH)
    fat = _pack(body_pose_w.T, global_orient_w.T, transl_w.T)
    betas_f, go_f, tr_f, bp_f = _run(idx2, betas_w.reshape(10), fat)
    return (betas_f.reshape(B, 10), go_f.reshape(B, DN),
            tr_f.reshape(B, DN), bp_f.reshape(B, DB))


# speedup vs baseline: 1.9833x; 1.0240x over previous
import functools

import jax
import jax.numpy as jnp
from jax import lax
from jax.experimental import pallas as pl
from jax.experimental.pallas import tpu as pltpu
from jax.experimental.pallas import tpu_sc as plsc

NUM_FRAMES = 100000
B = 16384
NC = 2
NS = 16
NW = NC * NS
BPW = B // NW
CH = 128
NCH = BPW // CH
L = 16
PERIOD = 80

DN = 3
DB = 69
FAT = 128
NV = BPW * DN
BV = BPW * DB

FBLK = 8192


def _dg(v, idx):
    dn = lax.GatherDimensionNumbers(
        offset_dims=(), collapsed_slice_dims=(0,), start_index_map=(0,))
    return lax.gather(v, idx[:, None], dn, slice_sizes=(1,),
                      mode=lax.GatherScatterMode.PROMISE_IN_BOUNDS)


def _pack_body(bp_ref, go_ref, tr_ref, out_ref):
    out_ref[:, 0:DB] = bp_ref[...].T
    gt = jnp.concatenate([go_ref[...], tr_ref[...]], axis=0)
    out_ref[:, DB:DB + 2 * DN] = gt.T


def _pack(bp_t, go_t, tr_t):
    grid = (NUM_FRAMES + FBLK - 1) // FBLK
    return pl.pallas_call(
        _pack_body,
        grid=(grid,),
        in_specs=[
            pl.BlockSpec((DB, FBLK), lambda n: (0, n)),
            pl.BlockSpec((DN, FBLK), lambda n: (0, n)),
            pl.BlockSpec((DN, FBLK), lambda n: (0, n)),
        ],
        out_specs=pl.BlockSpec((FBLK, FAT), lambda n: (n, 0)),
        out_shape=jax.ShapeDtypeStruct((NUM_FRAMES, FAT), jnp.float32),
    )(bp_t, go_t, tr_t)


def _body(idx_hbm, betas_hbm, fat_hbm,
          betas_out, go_out, tr_out, bp_out,
          idx_v, win0, win1, go_v, tr_v, bp_v, beta_row, beta_v,
          sem0, sem1):
    c = lax.axis_index("c")
    s = lax.axis_index("s")
    wid = s * NC + c
    lane = lax.iota(jnp.int32, L)

    pltpu.sync_copy(idx_hbm.at[pl.ds(wid * NCH, NCH)], idx_v)

    wins = (win0, win1)
    sems = (sem0, sem1)

    def fire(j):
        return pltpu.async_copy(fat_hbm.at[idx_v.at[j]], wins[j & 1],
                                sems[j & 1])

    inflight = fire(0)

    pltpu.sync_copy(betas_hbm, beta_row.at[pl.ds(0, 10)])
    row_v = beta_row[...]
    vregs = [_dg(row_v, lax.rem(lane + (k * L) % 10, 10))
             for k in range(PERIOD // L)]

    def bfill(i, carry):
        off = i * PERIOD
        for k in range(PERIOD // L):
            beta_v[pl.ds(off + k * L, L)] = vregs[k]
        return carry

    lax.fori_loop(0, BPW * 10 // PERIOD, bfill, 0)
    pltpu.sync_copy(beta_v, betas_out.at[pl.ds(wid * BPW * 10, BPW * 10)])

    rix_go = (lane + DB - 4 * L) & (L - 1)
    rix_tr = (lane + DB + DN - 4 * L) & (L - 1)

    for j in range(NCH):
        nxt = fire(j + 1) if j + 1 < NCH else None
        inflight.wait()
        inflight = nxt
        win = wins[j & 1]

        def ext(t, carry):
            k3 = DN * (j * CH) + DN * t
            k69 = DB * (j * CH) + DB * t
            w4 = win[t, pl.ds(4 * L, L)]
            for m in range(4):
                bp_v[pl.ds(k69 + m * L, L)] = win[t, pl.ds(m * L, L)]
            bp_v[pl.ds(k69 + 4 * L, L)] = w4
            go_v[pl.ds(k3, L)] = _dg(w4, rix_go)
            tr_v[pl.ds(k3, L)] = _dg(w4, rix_tr)
            return carry

        lax.fori_loop(0, CH, ext, 0)

    pltpu.sync_copy(go_v.at[pl.ds(0, NV)], go_out.at[pl.ds(wid * NV, NV)])
    pltpu.sync_copy(tr_v.at[pl.ds(0, NV)], tr_out.at[pl.ds(wid * NV, NV)])
    pltpu.sync_copy(bp_v.at[pl.ds(0, BV)], bp_out.at[pl.ds(wid * BV, BV)])


@jax.jit
def _run(idx2, betas_w, fat):
    mesh = plsc.VectorSubcoreMesh(core_axis_name="c", subcore_axis_name="s")
    f = pl.kernel(
        _body,
        mesh=mesh,
        compiler_params=pltpu.CompilerParams(use_tc_tiling_on_sc=False),
        out_type=(
            jax.ShapeDtypeStruct((B * 10,), jnp.float32),
            jax.ShapeDtypeStruct((B * DN,), jnp.float32),
            jax.ShapeDtypeStruct((B * DN,), jnp.float32),
            jax.ShapeDtypeStruct((B * DB,), jnp.float32),
        ),
        scratch_types=[
            pltpu.VMEM((NCH, CH), jnp.int32),
            pltpu.VMEM((CH, FAT), jnp.float32),
            pltpu.VMEM((CH, FAT), jnp.float32),
            pltpu.VMEM((NV + L,), jnp.float32),
            pltpu.VMEM((NV + L,), jnp.float32),
            pltpu.VMEM((BV + L,), jnp.float32),
            pltpu.VMEM((L,), jnp.float32),
            pltpu.VMEM((BPW * 10,), jnp.float32),
            pltpu.SemaphoreType.DMA,
            pltpu.SemaphoreType.DMA,
        ],
    )
    return f(idx2, betas_w, fat)


def kernel(frame_ids, betas_w, global_orient_w, transl_w, body_pose_w):
    idx2 = frame_ids.astype(jnp.int32).reshape(NW * NCH, CH)
    fat = _pack(body_pose_w.T, global_orient_w.T, transl_w.T)
    betas_f, go_f, tr_f, bp_f = _run(idx2, betas_w.reshape(10), fat)
    return (betas_f.reshape(B, 10), go_f.reshape(B, DN),
            tr_f.reshape(B, DN), bp_f.reshape(B, DB))
